# Initial kernel scaffold; baseline (speedup 1.0000x reference)
import jax
import jax.numpy as jnp
from jax.experimental import pallas as pl


def kernel(x, weight, lora_A, lora_B):
    raise NotImplementedError("write your pallas kernel here")


# speedup vs baseline: 8.0734x; 8.0734x over previous
import functools

import jax
import jax.numpy as jnp
from jax import lax
from jax.experimental import pallas as pl
from jax.experimental.pallas import tpu as pltpu
from jax.experimental.pallas import tpu_sc as plsc

_SCALING = 2.0


def _fuse_body(w_ref, a_ref, b_ref, o_ref):
    o_ref[...] = w_ref[...] + lax.dot(
        a_ref[...], b_ref[...], preferred_element_type=jnp.float32
    ) * _SCALING


def _fused_table(weight, lora_A, lora_B):
    V, D = weight.shape
    R = lora_A.shape[1]
    BLK = 4000
    assert V % BLK == 0
    return pl.pallas_call(
        _fuse_body,
        grid=(V // BLK,),
        in_specs=[
            pl.BlockSpec((BLK, D), lambda i: (i, 0)),
            pl.BlockSpec((BLK, R), lambda i: (i, 0)),
            pl.BlockSpec((R, D), lambda i: (0, 0)),
        ],
        out_specs=pl.BlockSpec((BLK, D), lambda i: (i, 0)),
        out_shape=jax.ShapeDtypeStruct((V, D), jnp.float32),
    )(weight, lora_A, lora_B)


def _sc_gather(table, idx2d):
    info = plsc.get_sparse_core_info()
    NC, NS = info.num_cores, info.num_subcores
    NW = NC * NS
    n_chunks, CH = idx2d.shape
    assert n_chunks % NW == 0
    per_w = n_chunks // NW
    V, D = table.shape
    N = n_chunks * CH

    mesh = plsc.VectorSubcoreMesh(core_axis_name="c", subcore_axis_name="s")

    @functools.partial(
        pl.kernel,
        mesh=mesh,
        compiler_params=pltpu.CompilerParams(use_tc_tiling_on_sc=False),
        out_type=jax.ShapeDtypeStruct((N, D), jnp.float32),
        scratch_types=[
            pltpu.VMEM((per_w, CH), jnp.int32),
            pltpu.VMEM((CH, D), jnp.float32),
            pltpu.SemaphoreType.DMA,
        ],
    )
    def k(table_hbm, idx_hbm, out_hbm, idx_v, rows_v, sem):
        wid = lax.axis_index("s") * NC + lax.axis_index("c")
        chunk0 = wid * per_w
        pltpu.sync_copy(idx_hbm.at[pl.ds(chunk0, per_w)], idx_v)

        def body(j, carry):
            pltpu.async_copy(table_hbm.at[idx_v.at[j]], rows_v, sem).wait()
            pltpu.sync_copy(rows_v, out_hbm.at[pl.ds((chunk0 + j) * CH, CH)])
            return carry

        lax.fori_loop(0, per_w, body, 0)

    return k(table, idx2d)


def kernel(x, weight, lora_A, lora_B):
    B, H = x.shape
    V, D = weight.shape
    fused = _fused_table(weight, lora_A, lora_B)
    idx = x.astype(jnp.int32).reshape(-1, 128)
    out = _sc_gather(fused, idx)
    return out.reshape(B, H, D)


# speedup vs baseline: 14.5826x; 1.8063x over previous
import functools

import jax
import jax.numpy as jnp
from jax import lax
from jax.experimental import pallas as pl
from jax.experimental.pallas import tpu as pltpu
from jax.experimental.pallas import tpu_sc as plsc

_SCALING = 2.0


def _fuse_body(wt_ref, at_ref, bt_ref, o_ref):
    eye = jnp.eye(64, dtype=jnp.float32)
    ft = wt_ref[...] + lax.dot(
        bt_ref[...], at_ref[...], preferred_element_type=jnp.float32
    ) * _SCALING
    t = lax.dot_general(
        ft, eye, (((0,), (0,)), ((), ())),
        preferred_element_type=jnp.float32,
    )
    o_ref[:, 0:64] = t


def _fused_table_pad(wT, aT, bT):
    D, V = wT.shape
    R = aT.shape[0]
    BLK = 8192
    nblk = (V + BLK - 1) // BLK
    return pl.pallas_call(
        _fuse_body,
        grid=(nblk,),
        in_specs=[
            pl.BlockSpec((D, BLK), lambda i: (0, i)),
            pl.BlockSpec((R, BLK), lambda i: (0, i)),
            pl.BlockSpec((D, R), lambda i: (0, 0)),
        ],
        out_specs=pl.BlockSpec((BLK, 2 * D), lambda i: (i, 0)),
        out_shape=jax.ShapeDtypeStruct((V, 2 * D), jnp.float32),
    )(wT, aT, bT)


def _sc_gather(table, idx2d):
    info = plsc.get_sparse_core_info()
    NC, NS = info.num_cores, info.num_subcores
    NW = NC * NS
    n_chunks, CH = idx2d.shape
    assert n_chunks % NW == 0
    per_w = n_chunks // NW
    V, Dpad = table.shape
    D = 64
    N = n_chunks * CH

    mesh = plsc.VectorSubcoreMesh(core_axis_name="c", subcore_axis_name="s")

    @functools.partial(
        pl.kernel,
        mesh=mesh,
        compiler_params=pltpu.CompilerParams(use_tc_tiling_on_sc=False),
        out_type=jax.ShapeDtypeStruct((N, D), jnp.float32),
        scratch_types=[
            pltpu.VMEM((per_w, CH), jnp.int32),
            pltpu.VMEM((CH, Dpad), jnp.float32),
            pltpu.SemaphoreType.DMA,
        ],
    )
    def k(table_hbm, idx_hbm, out_hbm, idx_v, rows_v, sem):
        wid = lax.axis_index("s") * NC + lax.axis_index("c")
        chunk0 = wid * per_w
        pltpu.sync_copy(idx_hbm.at[pl.ds(chunk0, per_w)], idx_v)

        def body(j, carry):
            pltpu.async_copy(table_hbm.at[idx_v.at[j]], rows_v, sem).wait()
            pltpu.sync_copy(
                rows_v.at[:, pl.ds(0, D)],
                out_hbm.at[pl.ds((chunk0 + j) * CH, CH)],
            )
            return carry

        lax.fori_loop(0, per_w, body, 0)

    return k(table, idx2d)


def kernel(x, weight, lora_A, lora_B):
    B, H = x.shape
    V, D = weight.shape
    table = _fused_table_pad(weight.T, lora_A.T, lora_B.T)
    idx = x.astype(jnp.int32).reshape(-1, 128)
    out = _sc_gather(table, idx)
    return out.reshape(B, H, D)
